# baseline (speedup 1.0000x reference)
import functools
import math

import jax
import jax.numpy as jnp
from jax import lax
from jax.experimental import pallas as pl
from jax.experimental.pallas import tpu as pltpu
from jax.experimental.pallas import tpu_sc as plsc

_EPS = 1e-5
_NW = 32
_CHUNK = 1024
_SUB = 128
_DP = 56


def _make_repack(F, V, D):
    nvc = -(-(V - _CHUNK) // _CHUNK) + 1
    total = F * nvc
    per_w = -(-total // _NW)
    vmax = V - _CHUNK
    mesh = plsc.VectorSubcoreMesh(core_axis_name="c", subcore_axis_name="s")

    @functools.partial(
        pl.kernel,
        mesh=mesh,
        compiler_params=pltpu.CompilerParams(use_tc_tiling_on_sc=False),
        out_type=jax.ShapeDtypeStruct((F * V, _DP), jnp.float32),
        scratch_types=[
            pltpu.VMEM((_CHUNK, D), jnp.float32),
        ],
    )
    def repack_k(tab3, out, vbuf):
        wid = lax.axis_index("s") * 2 + lax.axis_index("c")

        def chunk_body(t, carry):
            c = jnp.minimum(wid + _NW * t, total - 1)
            f = c // nvc
            v0 = jnp.minimum((c % nvc) * _CHUNK, vmax)
            pltpu.sync_copy(tab3.at[f, pl.ds(v0, _CHUNK)], vbuf)
            row0 = f * V + v0
            pltpu.sync_copy(vbuf.at[:, pl.ds(0, 48)],
                            out.at[pl.ds(row0, _CHUNK), pl.ds(0, 48)])
            pltpu.sync_copy(vbuf.at[:, pl.ds(48, 2)],
                            out.at[pl.ds(row0, _CHUNK), pl.ds(48, 2)])
            return carry

        lax.fori_loop(0, per_w, chunk_body, 0)

    return repack_k


def _make_gather(B, F):
    BF = B * F
    per_w = BF // _NW
    nch = per_w // _CHUNK
    nsub = _CHUNK // _SUB
    mesh = plsc.VectorSubcoreMesh(core_axis_name="c", subcore_axis_name="s")

    @functools.partial(
        pl.kernel,
        mesh=mesh,
        compiler_params=pltpu.CompilerParams(use_tc_tiling_on_sc=False),
        out_type=jax.ShapeDtypeStruct((F, B, _DP), jnp.float32),
        scratch_types=[
            pltpu.VMEM((_CHUNK,), jnp.int32),
            pltpu.VMEM((_CHUNK,), jnp.int32),
            pltpu.VMEM((nsub, _SUB), jnp.int32),
            pltpu.VMEM((_CHUNK, _DP), jnp.float32),
            pltpu.SemaphoreType.DMA,
        ],
    )
    def gather_k(tab, xf, off, out, xv, ov, iv, rows, sem):
        wid = lax.axis_index("s") * 2 + lax.axis_index("c")
        base = wid * per_w

        def chunk_body(c, carry):
            cb = base + c * _CHUNK
            pltpu.sync_copy(xf.at[pl.ds(cb, _CHUNK)], xv)
            pltpu.sync_copy(off.at[pl.ds(cb, _CHUNK)], ov)
            for r in range(nsub):
                for k in range(_SUB // 16):
                    s = r * _SUB + k * 16
                    iv[r, pl.ds(k * 16, 16)] = (
                        xv[pl.ds(s, 16)] + ov[pl.ds(s, 16)]
                    )
            handles = [
                pltpu.async_copy(
                    tab.at[iv.at[r]], rows.at[pl.ds(r * _SUB, _SUB)], sem)
                for r in range(nsub)
            ]
            for h in handles:
                h.wait()
            f = cb // B
            b0 = cb % B
            pltpu.sync_copy(rows, out.at[f, pl.ds(b0, _CHUNK)])
            return carry

        lax.fori_loop(0, nch, chunk_body, 0)

    return gather_k


def _mlp(g3d, W1s, c1, W2f, c2, W3f, c3):
    F, B, _ = g3d.shape
    H = W1s.shape[2]
    BM = 512

    def body(g_ref, w1_ref, c1_ref, w2_ref, c2_ref, w3_ref, c3_ref, out_ref):
        col = lax.broadcasted_iota(jnp.int32, (BM, _DP), 1)
        mask = col < 50
        h1 = c1_ref[...]
        for f in range(F):
            gf = jnp.where(mask, g_ref[f], 0.0)
            h1 = h1 + jnp.dot(gf, w1_ref[f],
                              preferred_element_type=jnp.float32)
        h1 = jnp.maximum(h1, 0.0)
        h2 = jnp.maximum(
            jnp.dot(h1, w2_ref[...], preferred_element_type=jnp.float32)
            + c2_ref[...], 0.0)
        l = (jnp.dot(h2, w3_ref[...], preferred_element_type=jnp.float32)
             + c3_ref[...])
        m = jnp.max(l, axis=1, keepdims=True)
        e = jnp.exp(l - m)
        out_ref[...] = e / jnp.sum(e, axis=1, keepdims=True)

    return pl.pallas_call(
        body,
        grid=(B // BM,),
        in_specs=[
            pl.BlockSpec((F, BM, _DP), lambda i: (0, i, 0)),
            pl.BlockSpec((F, _DP, H), lambda i: (0, 0, 0)),
            pl.BlockSpec((1, H), lambda i: (0, 0)),
            pl.BlockSpec((H, H), lambda i: (0, 0)),
            pl.BlockSpec((1, H), lambda i: (0, 0)),
            pl.BlockSpec((H, 2), lambda i: (0, 0)),
            pl.BlockSpec((1, 2), lambda i: (0, 0)),
        ],
        out_specs=pl.BlockSpec((BM, 2), lambda i: (i, 0)),
        out_shape=jax.ShapeDtypeStruct((B, 2), jnp.float32),
    )(g3d, W1s, c1, W2f, c2, W3f, c3)


def kernel(x, tables, bn0_g, bn0_b, W1, b1, bn1_g, bn1_b, W2, b2, bn2_g, bn2_b, W3, b3):
    F, V, D = tables.shape
    B = x.shape[0]

    tab56 = _make_repack(F, V, D)(tables)
    xf = x.T.reshape(-1)
    off = jnp.repeat(jnp.arange(F, dtype=jnp.int32) * V, B)

    g3d = _make_gather(B, F)(tab56, xf, off)

    inv = 1.0 / math.sqrt(1.0 + _EPS)
    s0, t0 = bn0_g * inv, bn0_b
    s1, t1 = bn1_g * inv, bn1_b
    s2, t2 = bn2_g * inv, bn2_b
    H = W1.shape[1]
    W1s = (W1 * s0[:, None]).reshape(F, D, H)
    W1s = jnp.pad(W1s, ((0, 0), (0, _DP - D), (0, 0)))
    c1 = (t0 @ W1 + b1)[None, :]
    W2f = W2 * s1[:, None]
    c2 = (t1 @ W2 + b2)[None, :]
    W3f = W3 * s2[:, None]
    c3 = (t2 @ W3 + b3)[None, :]

    return _mlp(g3d, W1s, c1, W2f, c2, W3f, c3)


# speedup vs baseline: 1.3239x; 1.3239x over previous
import functools
import math

import jax
import jax.numpy as jnp
from jax import lax
from jax.experimental import pallas as pl
from jax.experimental.pallas import tpu as pltpu
from jax.experimental.pallas import tpu_sc as plsc

_EPS = 1e-5
_NW = 32
_CHUNK = 1024
_SUB = 128
_DP = 56


def _make_gather(B, F):
    BF = B * F
    per_w = BF // _NW
    nch = per_w // _CHUNK
    nsub = _CHUNK // _SUB
    mesh = plsc.VectorSubcoreMesh(core_axis_name="c", subcore_axis_name="s")

    @functools.partial(
        pl.kernel,
        mesh=mesh,
        compiler_params=pltpu.CompilerParams(use_tc_tiling_on_sc=False),
        out_type=jax.ShapeDtypeStruct((BF, _DP), jnp.float32),
        scratch_types=[
            pltpu.VMEM((_CHUNK,), jnp.int32),
            pltpu.VMEM((_CHUNK,), jnp.int32),
            pltpu.VMEM((nsub, _SUB), jnp.int32),
            pltpu.VMEM((_CHUNK, _DP), jnp.float32),
            pltpu.SemaphoreType.DMA,
        ],
    )
    def gather_k(tab, xf, off, out, xv, ov, iv, rows, sem):
        wid = lax.axis_index("s") * 2 + lax.axis_index("c")
        base = wid * per_w

        def chunk_body(c, carry):
            cb = base + c * _CHUNK
            pltpu.sync_copy(xf.at[pl.ds(cb, _CHUNK)], xv)
            pltpu.sync_copy(off.at[pl.ds(cb, _CHUNK)], ov)
            for r in range(nsub):
                for k in range(_SUB // 16):
                    s = r * _SUB + k * 16
                    iv[r, pl.ds(k * 16, 16)] = (
                        xv[pl.ds(s, 16)] + ov[pl.ds(s, 16)]
                    )
            handles = [
                pltpu.async_copy(
                    tab.at[iv.at[r]], rows.at[pl.ds(r * _SUB, _SUB)], sem)
                for r in range(nsub)
            ]
            for h in handles:
                h.wait()
            pltpu.sync_copy(rows, out.at[pl.ds(cb, _CHUNK)])
            return carry

        lax.fori_loop(0, nch, chunk_body, 0)

    return gather_k


def _mlp(grows, B, F, W1f, c1, W2f, c2, W3f, c3):
    CAT = F * _DP
    H = W1f.shape[2]
    BM = 512

    def body(g_hbm, w1_ref, c1_ref, w2_ref, c2_ref, w3_ref, c3_ref, out_ref,
             gbuf, sem):
        i = pl.program_id(0)
        copies = [
            pltpu.make_async_copy(
                g_hbm.at[pl.ds(f * B + i * BM, BM)],
                gbuf.at[f], sem)
            for f in range(F)
        ]
        for cp in copies:
            cp.start()
        for cp in copies:
            cp.wait()
        col = lax.broadcasted_iota(jnp.int32, (BM, _DP), 1)
        mask = col < 50
        h1 = c1_ref[...]
        for f in range(F):
            gf = jnp.where(mask, gbuf[f], 0.0)
            h1 = h1 + jnp.dot(gf, w1_ref[f],
                              preferred_element_type=jnp.float32)
        h1 = jnp.maximum(h1, 0.0)
        h2 = jnp.maximum(
            jnp.dot(h1, w2_ref[...], preferred_element_type=jnp.float32)
            + c2_ref[...], 0.0)
        l = (jnp.dot(h2, w3_ref[...], preferred_element_type=jnp.float32)
             + c3_ref[...])
        m = jnp.max(l, axis=1, keepdims=True)
        e = jnp.exp(l - m)
        out_ref[...] = e / jnp.sum(e, axis=1, keepdims=True)

    return pl.pallas_call(
        body,
        grid=(B // BM,),
        in_specs=[
            pl.BlockSpec(memory_space=pl.ANY),
            pl.BlockSpec((F, _DP, H), lambda i: (0, 0, 0)),
            pl.BlockSpec((1, H), lambda i: (0, 0)),
            pl.BlockSpec((H, H), lambda i: (0, 0)),
            pl.BlockSpec((1, H), lambda i: (0, 0)),
            pl.BlockSpec((H, 2), lambda i: (0, 0)),
            pl.BlockSpec((1, 2), lambda i: (0, 0)),
        ],
        out_specs=pl.BlockSpec((BM, 2), lambda i: (i, 0)),
        out_shape=jax.ShapeDtypeStruct((B, 2), jnp.float32),
        scratch_shapes=[
            pltpu.VMEM((F, BM, _DP), jnp.float32),
            pltpu.SemaphoreType.DMA,
        ],
    )(grows, W1f, c1, W2f, c2, W3f, c3)


def kernel(x, tables, bn0_g, bn0_b, W1, b1, bn1_g, bn1_b, W2, b2, bn2_g, bn2_b, W3, b3):
    F, V, D = tables.shape
    B = x.shape[0]

    tab56 = jnp.pad(tables, ((0, 0), (0, 0), (0, _DP - D))).reshape(F * V, _DP)
    xf = x.T.reshape(-1)
    off = jnp.repeat(jnp.arange(F, dtype=jnp.int32) * V, B)

    grows = _make_gather(B, F)(tab56, xf, off)

    inv = 1.0 / math.sqrt(1.0 + _EPS)
    s0, t0 = bn0_g * inv, bn0_b
    s1, t1 = bn1_g * inv, bn1_b
    s2, t2 = bn2_g * inv, bn2_b
    H = W1.shape[1]
    W1f = (W1 * s0[:, None]).reshape(F, D, H)
    W1f = jnp.pad(W1f, ((0, 0), (0, _DP - D), (0, 0)))
    c1 = (t0 @ W1 + b1)[None, :]
    W2f = W2 * s1[:, None]
    c2 = (t1 @ W2 + b2)[None, :]
    W3f = W3 * s2[:, None]
    c3 = (t2 @ W3 + b3)[None, :]

    return _mlp(grows, B, F, W1f, c1, W2f, c2, W3f, c3)


# speedup vs baseline: 2.8393x; 2.1446x over previous
import functools
import math

import jax
import jax.numpy as jnp
from jax import lax
from jax.experimental import pallas as pl
from jax.experimental.pallas import tpu as pltpu
from jax.experimental.pallas import tpu_sc as plsc

_EPS = 1e-5
_NW = 32
_CHUNK = 1024
_SUB = 128
_DP = 56


def _make_gather(B, F):
    BF = B * F
    per_w = BF // _NW
    nch = per_w // _CHUNK
    nsub = _CHUNK // _SUB
    mesh = plsc.VectorSubcoreMesh(core_axis_name="c", subcore_axis_name="s")

    @functools.partial(
        pl.kernel,
        mesh=mesh,
        compiler_params=pltpu.CompilerParams(use_tc_tiling_on_sc=False),
        out_type=jax.ShapeDtypeStruct((BF, _DP), jnp.float32),
        scratch_types=[
            pltpu.VMEM((_CHUNK,), jnp.int32),
            pltpu.VMEM((_CHUNK,), jnp.int32),
            pltpu.VMEM((nsub, _SUB), jnp.int32),
            pltpu.VMEM((_CHUNK, _DP), jnp.float32),
            pltpu.SemaphoreType.DMA,
        ],
    )
    def gather_k(tab, xf, off, out, xv, ov, iv, rows, sem):
        wid = lax.axis_index("s") * 2 + lax.axis_index("c")
        base = wid * per_w

        def chunk_body(c, carry):
            cb = base + c * _CHUNK
            pltpu.sync_copy(xf.at[pl.ds(cb, _CHUNK)], xv)
            pltpu.sync_copy(off.at[pl.ds(cb, _CHUNK)], ov)
            for r in range(nsub):
                for k in range(_SUB // 16):
                    s = r * _SUB + k * 16
                    iv[r, pl.ds(k * 16, 16)] = (
                        xv[pl.ds(s, 16)] + ov[pl.ds(s, 16)]
                    )
            handles = [
                pltpu.async_copy(
                    tab.at[iv.at[r]], rows.at[pl.ds(r * _SUB, _SUB)], sem)
                for r in range(nsub)
            ]
            for h in handles:
                h.wait()
            pltpu.sync_copy(rows, out.at[pl.ds(cb, _CHUNK)])
            return carry

        lax.fori_loop(0, nch, chunk_body, 0)

    return gather_k


def _mlp(grows, B, F, W1f, c1, W2f, c2, W3f, c3):
    CAT = F * _DP
    H = W1f.shape[2]
    BM = 512

    def body(g_hbm, w1_ref, c1_ref, w2_ref, c2_ref, w3_ref, c3_ref, out_ref,
             gbuf, sem):
        i = pl.program_id(0)
        copies = [
            pltpu.make_async_copy(
                g_hbm.at[pl.ds(f * B + i * BM, BM)],
                gbuf.at[f], sem)
            for f in range(F)
        ]
        for cp in copies:
            cp.start()
        for cp in copies:
            cp.wait()
        col = lax.broadcasted_iota(jnp.int32, (BM, _DP), 1)
        mask = col < 50
        h1 = c1_ref[...]
        for f in range(F):
            gf = jnp.where(mask, gbuf[f], 0.0)
            h1 = h1 + jnp.dot(gf, w1_ref[f],
                              preferred_element_type=jnp.float32)
        h1 = jnp.maximum(h1, 0.0)
        h2 = jnp.maximum(
            jnp.dot(h1, w2_ref[...], preferred_element_type=jnp.float32)
            + c2_ref[...], 0.0)
        l = (jnp.dot(h2, w3_ref[...], preferred_element_type=jnp.float32)
             + c3_ref[...])
        m = jnp.max(l, axis=1, keepdims=True)
        e = jnp.exp(l - m)
        out_ref[...] = e / jnp.sum(e, axis=1, keepdims=True)

    return pl.pallas_call(
        body,
        grid=(B // BM,),
        in_specs=[
            pl.BlockSpec(memory_space=pl.ANY),
            pl.BlockSpec((F, _DP, H), lambda i: (0, 0, 0)),
            pl.BlockSpec((1, H), lambda i: (0, 0)),
            pl.BlockSpec((H, H), lambda i: (0, 0)),
            pl.BlockSpec((1, H), lambda i: (0, 0)),
            pl.BlockSpec((H, 2), lambda i: (0, 0)),
            pl.BlockSpec((1, 2), lambda i: (0, 0)),
        ],
        out_specs=pl.BlockSpec((BM, 2), lambda i: (i, 0)),
        out_shape=jax.ShapeDtypeStruct((B, 2), jnp.float32),
        scratch_shapes=[
            pltpu.VMEM((F, BM, _DP), jnp.float32),
            pltpu.SemaphoreType.DMA,
        ],
    )(grows, W1f, c1, W2f, c2, W3f, c3)


def kernel(x, tables, bn0_g, bn0_b, W1, b1, bn1_g, bn1_b, W2, b2, bn2_g, bn2_b, W3, b3):
    F, V, D = tables.shape
    B = x.shape[0]

    tab56 = jnp.pad(tables.reshape(F * V, D), ((0, 0), (0, _DP - D)))
    xf = x.T.reshape(-1)
    off = jnp.repeat(jnp.arange(F, dtype=jnp.int32) * V, B)

    grows = _make_gather(B, F)(tab56, xf, off)

    inv = 1.0 / math.sqrt(1.0 + _EPS)
    s0, t0 = bn0_g * inv, bn0_b
    s1, t1 = bn1_g * inv, bn1_b
    s2, t2 = bn2_g * inv, bn2_b
    H = W1.shape[1]
    W1f = (W1 * s0[:, None]).reshape(F, D, H)
    W1f = jnp.pad(W1f, ((0, 0), (0, _DP - D), (0, 0)))
    c1 = (t0 @ W1 + b1)[None, :]
    W2f = W2 * s1[:, None]
    c2 = (t1 @ W2 + b2)[None, :]
    W3f = W3 * s2[:, None]
    c3 = (t2 @ W3 + b3)[None, :]

    return _mlp(grows, B, F, W1f, c1, W2f, c2, W3f, c3)


# speedup vs baseline: 3.0130x; 1.0612x over previous
import functools
import math

import jax
import jax.numpy as jnp
from jax import lax
from jax.experimental import pallas as pl
from jax.experimental.pallas import tpu as pltpu
from jax.experimental.pallas import tpu_sc as plsc

_EPS = 1e-5
_NW = 32
_CHUNK = 1024
_SUB = 128
_DP = 56


def _make_gather(BF):
    per_w = BF // _NW
    nch = per_w // _CHUNK
    nsub = _CHUNK // _SUB
    mesh = plsc.VectorSubcoreMesh(core_axis_name="c", subcore_axis_name="s")

    @functools.partial(
        pl.kernel,
        mesh=mesh,
        compiler_params=pltpu.CompilerParams(use_tc_tiling_on_sc=False),
        out_type=jax.ShapeDtypeStruct((BF, _DP), jnp.float32),
        scratch_types=[
            pltpu.VMEM((_CHUNK,), jnp.int32),
            pltpu.VMEM((_CHUNK,), jnp.int32),
            pltpu.VMEM((nsub, _SUB), jnp.int32),
            pltpu.VMEM((_CHUNK, _DP), jnp.float32),
            pltpu.SemaphoreType.DMA,
        ],
    )
    def gather_k(tab, xf, off, out, xv, ov, iv, rows, sem):
        wid = lax.axis_index("s") * 2 + lax.axis_index("c")
        base = wid * per_w

        def chunk_body(c, carry):
            cb = base + c * _CHUNK
            pltpu.sync_copy(xf.at[pl.ds(cb, _CHUNK)], xv)
            pltpu.sync_copy(off.at[pl.ds(cb, _CHUNK)], ov)
            for r in range(nsub):
                for k in range(_SUB // 16):
                    s = r * _SUB + k * 16
                    iv[r, pl.ds(k * 16, 16)] = (
                        xv[pl.ds(s, 16)] + ov[pl.ds(s, 16)]
                    )
            handles = [
                pltpu.async_copy(
                    tab.at[iv.at[r]], rows.at[pl.ds(r * _SUB, _SUB)], sem)
                for r in range(nsub)
            ]
            for h in handles:
                h.wait()
            pltpu.sync_copy(rows, out.at[pl.ds(cb, _CHUNK)])
            return carry

        lax.fori_loop(0, nch, chunk_body, 0)

    return gather_k


def _mlp(g2d, W1f, c1, W2f, c2, W3f, c3):
    B, CAT = g2d.shape
    H = W1f.shape[1]
    BM = 512

    def body(g_ref, w1_ref, c1_ref, w2_ref, c2_ref, w3_ref, c3_ref, out_ref):
        h1 = jnp.maximum(
            jnp.dot(g_ref[...], w1_ref[...], preferred_element_type=jnp.float32)
            + c1_ref[...], 0.0)
        h2 = jnp.maximum(
            jnp.dot(h1, w2_ref[...], preferred_element_type=jnp.float32)
            + c2_ref[...], 0.0)
        l = (jnp.dot(h2, w3_ref[...], preferred_element_type=jnp.float32)
             + c3_ref[...])
        m = jnp.max(l, axis=1, keepdims=True)
        e = jnp.exp(l - m)
        out_ref[...] = e / jnp.sum(e, axis=1, keepdims=True)

    return pl.pallas_call(
        body,
        grid=(B // BM,),
        in_specs=[
            pl.BlockSpec((BM, CAT), lambda i: (i, 0)),
            pl.BlockSpec((CAT, H), lambda i: (0, 0)),
            pl.BlockSpec((1, H), lambda i: (0, 0)),
            pl.BlockSpec((H, H), lambda i: (0, 0)),
            pl.BlockSpec((1, H), lambda i: (0, 0)),
            pl.BlockSpec((H, 2), lambda i: (0, 0)),
            pl.BlockSpec((1, 2), lambda i: (0, 0)),
        ],
        out_specs=pl.BlockSpec((BM, 2), lambda i: (i, 0)),
        out_shape=jax.ShapeDtypeStruct((B, 2), jnp.float32),
    )(g2d, W1f, c1, W2f, c2, W3f, c3)


def kernel(x, tables, bn0_g, bn0_b, W1, b1, bn1_g, bn1_b, W2, b2, bn2_g, bn2_b, W3, b3):
    F, V, D = tables.shape
    B = x.shape[0]
    BF = B * F

    tab56 = jnp.pad(tables.reshape(F * V, D), ((0, 0), (0, _DP - D)))
    xf = x.reshape(-1)
    off = jnp.tile(jnp.arange(F, dtype=jnp.int32) * V, B)

    gathered = _make_gather(BF)(tab56, xf, off)
    g2d = gathered.reshape(B, F * _DP)

    inv = 1.0 / math.sqrt(1.0 + _EPS)
    s0, t0 = bn0_g * inv, bn0_b
    s1, t1 = bn1_g * inv, bn1_b
    s2, t2 = bn2_g * inv, bn2_b
    H = W1.shape[1]
    W1f = (W1 * s0[:, None]).reshape(F, D, H)
    W1f = jnp.pad(W1f, ((0, 0), (0, _DP - D), (0, 0))).reshape(F * _DP, H)
    c1 = (t0 @ W1 + b1)[None, :]
    W2f = W2 * s1[:, None]
    c2 = (t1 @ W2 + b2)[None, :]
    W3f = W3 * s2[:, None]
    c3 = (t2 @ W3 + b3)[None, :]

    return _mlp(g2d, W1f, c1, W2f, c2, W3f, c3)
